# Initial kernel scaffold; baseline (speedup 1.0000x reference)
import jax
import jax.numpy as jnp
from jax.experimental import pallas as pl


def kernel(coords, features, Wmlp1, bmlp1, Wmlp2, bmlp2, gmlp, bmlpln, Wse1, bse1, Wse2, bse2, gse, bseln, Wr, Wpos, bpos, Wq, bq, Wk, bk, Wv, bv, Wo, bo, g1, b1n, Wff1, bff1, Wff2, bff2, g2, b2n):
    raise NotImplementedError("write your pallas kernel here")


# speedup vs baseline: 5.2141x; 5.2141x over previous
import functools

import jax
import jax.numpy as jnp
from jax import lax
from jax.experimental import pallas as pl
from jax.experimental.pallas import tpu as pltpu
from jax.experimental.pallas import tpu_sc as plsc

L, N, D_IN, D, F, KNN, H, REPEATS = 1024, 4, 64, 256, 128, 16, 4, 2
B = L * N
DH = D // H
FH = F // 2
TW = 640
TILE_D = 128


def _mish(x):
    sp = jnp.maximum(x, 0.0) + jnp.log1p(jnp.exp(-jnp.abs(x)))
    return x * jnp.tanh(sp)


def _ln(x, g, b, eps=1e-5):
    m = x.mean(-1, keepdims=True)
    v = ((x - m) ** 2).mean(-1, keepdims=True)
    return (x - m) / jnp.sqrt(v + eps) * g + b


def _featk(feats_ref, c8_ref, mmean_ref, nmask_ref,
           w1_ref, b1_ref, w2_ref, b2_ref, gm_ref, bm_ref,
           ws1_ref, bs1_ref, ws2_ref, bs2_ref, gs_ref, bs_ref,
           wk_ref, wv_ref, wr8_ref,
           feat_out, table_out, pw_out):
    x = feats_ref[:]
    h = _mish(x @ w1_ref[:] + b1_ref[:]) @ w2_ref[:] + b2_ref[:]
    feat = _ln(h, gm_ref[:], bm_ref[:])
    s4 = mmean_ref[:] @ feat
    s4 = jax.nn.sigmoid(
        jnp.maximum(s4 @ ws1_ref[:] + bs1_ref[:], 0.0) @ ws2_ref[:] + bs2_ref[:])
    srow = nmask_ref[:] @ s4
    feat = _ln(feat + feat * srow, gs_ref[:], bs_ref[:])
    fk = feat @ wk_ref[:]
    fv = feat @ wv_ref[:]
    pw = c8_ref[:] @ wr8_ref[:]
    feat_out[:] = feat
    pw_out[:] = pw
    table_out[:] = jnp.concatenate(
        [fk, fv, pw, jnp.zeros((B, TW - 2 * D - FH), jnp.float32)], axis=1)


def _topk(c8_ref, cT_ref, idx_out):
    a = c8_ref[0]
    t = cT_ref[0]
    d2 = ((a[:, 0:1] - t[0:1, :]) ** 2
          + (a[:, 1:2] - t[1:2, :]) ** 2
          + (a[:, 2:3] - t[2:3, :]) ** 2)
    iot = lax.broadcasted_iota(jnp.int32, (L, L), 1)
    cols = []
    for _ in range(KNN):
        m = jnp.min(d2, axis=1, keepdims=True)
        am = jnp.min(jnp.where(d2 == m, iot, L), axis=1, keepdims=True)
        cols.append(am)
        d2 = jnp.where(iot == am, jnp.float32(1e30), d2)
    idx_out[0] = jnp.concatenate(cols, axis=1)


def _sc_gather(table, gidx):
    info = plsc.get_sparse_core_info()
    nc, ns = info.num_cores, info.num_subcores
    nw = nc * ns
    rows_w = (KNN * B) // nw
    chunk = 64
    steps = rows_w // chunk

    mesh = plsc.VectorSubcoreMesh(core_axis_name="c", subcore_axis_name="s")

    @functools.partial(
        pl.kernel, mesh=mesh,
        out_type=jax.ShapeDtypeStruct((KNN * B, TW), jnp.float32),
        scratch_types=[
            pltpu.VMEM((chunk,), jnp.int32),
            pltpu.VMEM((chunk, TW), jnp.float32),
            pltpu.SemaphoreType.DMA,
        ],
    )
    def k(table_hbm, gidx_hbm, out_hbm, idx_v, rows_v, sem):
        wid = lax.axis_index("s") * nc + lax.axis_index("c")
        base = wid * rows_w

        def step(i, carry):
            off = base + i * chunk
            pltpu.sync_copy(gidx_hbm.at[pl.ds(off, chunk)], idx_v)
            pltpu.async_copy(table_hbm.at[idx_v], rows_v, sem).wait()
            pltpu.sync_copy(rows_v, out_hbm.at[pl.ds(off, chunk)])
            return carry

        lax.fori_loop(0, steps, step, 0)

    return k(table, gidx)


def _attn(g_ref, feat_ref, pw_ref, seg_ref, segT_ref,
          wposk_ref, wposv_ref, bkk_ref, bvv_ref,
          wq_ref, bq_ref, wo_ref, bo_ref, g1_ref, b1_ref,
          wf1_ref, bf1_ref, wf2_ref, bf2_ref, g2_ref, b2_ref,
          out_ref):
    T = TILE_D
    g = g_ref[:]
    fk = g[:, :, 0:D].reshape(KNN * T, D)
    fv = g[:, :, D:2 * D].reshape(KNN * T, D)
    pwg = g[:, :, 2 * D:2 * D + FH]
    proj = pwg - pw_ref[:][None, :, :]
    tp = (2.0 * jnp.pi) * proj
    ffeat = (jnp.concatenate([jnp.cos(tp), jnp.sin(tp)], axis=-1)
             / jnp.sqrt(jnp.float32(F))).reshape(KNN * T, F)
    kk = (fk + ffeat @ wposk_ref[:] + bkk_ref[:]).reshape(KNN, T, D)
    vv = (fv + ffeat @ wposv_ref[:] + bvv_ref[:]).reshape(KNN, T, D)

    tgt = feat_ref[:]
    inv = 1.0 / jnp.sqrt(jnp.float32(DH))
    for _ in range(REPEATS):
        q = tgt @ wq_ref[:] + bq_ref[:]
        sc = ((kk * q[None, :, :]).reshape(KNN * T, D)
              @ seg_ref[:]).reshape(KNN, T, H) * inv
        m = jnp.max(sc, axis=0, keepdims=True)
        e = jnp.exp(sc - m)
        a = e / jnp.sum(e, axis=0, keepdims=True)
        ab = (a.reshape(KNN * T, H) @ segT_ref[:]).reshape(KNN, T, D)
        att = jnp.sum(ab * vv, axis=0)
        att = att @ wo_ref[:] + bo_ref[:]
        tgt = _ln(tgt + att, g1_ref[:], b1_ref[:])
        ffn = _mish(tgt @ wf1_ref[:] + bf1_ref[:]) @ wf2_ref[:] + bf2_ref[:]
        tgt = _ln(tgt + ffn, g2_ref[:], b2_ref[:])
    out_ref[:] = tgt


def kernel(coords, features, Wmlp1, bmlp1, Wmlp2, bmlp2, gmlp, bmlpln,
           Wse1, bse1, Wse2, bse2, gse, bseln, Wr, Wpos, bpos,
           Wq, bq, Wk, bk, Wv, bv, Wo, bo, g1, b1n,
           Wff1, bff1, Wff2, bff2, g2, b2n):
    f32 = jnp.float32
    r2 = lambda v: v.reshape(1, -1)

    feats_flat = features.reshape(B, D_IN)
    c_flat = coords.reshape(B, 3)
    c8 = jnp.pad(c_flat, ((0, 0), (0, 5)))
    wr8 = jnp.pad(Wr, ((0, 5), (0, 0)))
    cn8 = jnp.pad(jnp.transpose(coords, (1, 0, 2)), ((0, 0), (0, 0), (0, 5)))
    cnT = jnp.transpose(cn8, (0, 2, 1))
    nvec = jnp.arange(B, dtype=jnp.int32) % N
    nmask = (nvec[:, None] == jnp.arange(N)[None, :]).astype(f32)
    mmean = nmask.T / f32(L)
    seg = (jnp.arange(D)[:, None] // DH
           == jnp.arange(H)[None, :]).astype(f32)
    segT = seg.T
    wposk = Wpos @ Wk
    wposv = Wpos @ Wv
    bkk = bpos @ Wk + bk
    bvv = bpos @ Wv + bv

    feat, table, pw = pl.pallas_call(
        _featk,
        out_shape=(
            jax.ShapeDtypeStruct((B, D), f32),
            jax.ShapeDtypeStruct((B, TW), f32),
            jax.ShapeDtypeStruct((B, FH), f32),
        ),
    )(feats_flat, c8, mmean, nmask,
      Wmlp1, r2(bmlp1), Wmlp2, r2(bmlp2), r2(gmlp), r2(bmlpln),
      Wse1, r2(bse1), Wse2, r2(bse2), r2(gse), r2(bseln),
      Wk, Wv, wr8)

    idx16 = pl.pallas_call(
        _topk,
        grid=(N,),
        in_specs=[
            pl.BlockSpec((1, L, 8), lambda n: (n, 0, 0)),
            pl.BlockSpec((1, 8, L), lambda n: (n, 0, 0)),
        ],
        out_specs=pl.BlockSpec((1, L, KNN), lambda n: (n, 0, 0)),
        out_shape=jax.ShapeDtypeStruct((N, L, KNN), jnp.int32),
    )(cn8, cnT)

    gidx = (jnp.transpose(idx16, (2, 1, 0)) * N
            + jnp.arange(N, dtype=jnp.int32)[None, None, :]).reshape(KNN * B)

    rows = _sc_gather(table, gidx).reshape(KNN, B, TW)

    nsteps = B // TILE_D
    out = pl.pallas_call(
        _attn,
        grid=(nsteps,),
        in_specs=[
            pl.BlockSpec((KNN, TILE_D, TW), lambda i: (0, i, 0)),
            pl.BlockSpec((TILE_D, D), lambda i: (i, 0)),
            pl.BlockSpec((TILE_D, FH), lambda i: (i, 0)),
            pl.BlockSpec((D, H), lambda i: (0, 0)),
            pl.BlockSpec((H, D), lambda i: (0, 0)),
            pl.BlockSpec((F, D), lambda i: (0, 0)),
            pl.BlockSpec((F, D), lambda i: (0, 0)),
            pl.BlockSpec((1, D), lambda i: (0, 0)),
            pl.BlockSpec((1, D), lambda i: (0, 0)),
            pl.BlockSpec((D, D), lambda i: (0, 0)),
            pl.BlockSpec((1, D), lambda i: (0, 0)),
            pl.BlockSpec((D, D), lambda i: (0, 0)),
            pl.BlockSpec((1, D), lambda i: (0, 0)),
            pl.BlockSpec((1, D), lambda i: (0, 0)),
            pl.BlockSpec((1, D), lambda i: (0, 0)),
            pl.BlockSpec((D, 4 * D), lambda i: (0, 0)),
            pl.BlockSpec((1, 4 * D), lambda i: (0, 0)),
            pl.BlockSpec((4 * D, D), lambda i: (0, 0)),
            pl.BlockSpec((1, D), lambda i: (0, 0)),
            pl.BlockSpec((1, D), lambda i: (0, 0)),
            pl.BlockSpec((1, D), lambda i: (0, 0)),
        ],
        out_specs=pl.BlockSpec((TILE_D, D), lambda i: (i, 0)),
        out_shape=jax.ShapeDtypeStruct((B, D), f32),
    )(rows, feat, pw, seg, segT, wposk, wposv, r2(bkk), r2(bvv),
      Wq, r2(bq), Wo, r2(bo), r2(g1), r2(b1n),
      Wff1, r2(bff1), Wff2, r2(bff2), r2(g2), r2(b2n))

    return out.reshape(L, N, D)


# speedup vs baseline: 5.5802x; 1.0702x over previous
import functools

import jax
import jax.numpy as jnp
from jax import lax
from jax.experimental import pallas as pl
from jax.experimental.pallas import tpu as pltpu
from jax.experimental.pallas import tpu_sc as plsc

L, N, D_IN, D, F, KNN, H, REPEATS = 1024, 4, 64, 256, 128, 16, 4, 2
B = L * N
DH = D // H
FH = F // 2
TW = 384
TILE_D = 128


def _mm(a, b):
    return lax.dot(a.astype(jnp.bfloat16), b.astype(jnp.bfloat16),
                   preferred_element_type=jnp.float32)


def _mish(x):
    sp = jnp.maximum(x, 0.0) + jnp.log1p(jnp.exp(-jnp.abs(x)))
    return x * jnp.tanh(sp)


def _ln(x, g, b, eps=1e-5):
    m = x.mean(-1, keepdims=True)
    v = ((x - m) ** 2).mean(-1, keepdims=True)
    return (x - m) / jnp.sqrt(v + eps) * g + b


def _featk(feats_ref, c8_ref, mmean_ref, nmask_ref,
           w1_ref, b1_ref, w2_ref, b2_ref, gm_ref, bm_ref,
           ws1_ref, bs1_ref, ws2_ref, bs2_ref, gs_ref, bs_ref,
           wk_ref, wv_ref, wr8_ref,
           feat_out, table_out, pw_out):
    x = feats_ref[:]
    h = _mish(x @ w1_ref[:] + b1_ref[:]) @ w2_ref[:] + b2_ref[:]
    feat = _ln(h, gm_ref[:], bm_ref[:])
    s4 = mmean_ref[:] @ feat
    s4 = jax.nn.sigmoid(
        jnp.maximum(s4 @ ws1_ref[:] + bs1_ref[:], 0.0) @ ws2_ref[:] + bs2_ref[:])
    srow = nmask_ref[:] @ s4
    feat = _ln(feat + feat * srow, gs_ref[:], bs_ref[:])
    fk = feat @ wk_ref[:]
    fv = feat @ wv_ref[:]
    pw = c8_ref[:] @ wr8_ref[:]
    feat_out[:] = feat
    pw_out[:] = pw
    uk = lax.bitcast_convert_type(fk, jnp.uint32)
    uv = lax.bitcast_convert_type(fv, jnp.uint32)
    packed = lax.bitcast_convert_type(
        (uk & jnp.uint32(0xFFFF0000)) | (uv >> 16), jnp.float32)
    table_out[:] = jnp.concatenate(
        [packed, pw, jnp.zeros((B, TW - D - FH), jnp.float32)], axis=1)


def _topk(c8_ref, cT_ref, idx_out):
    a = c8_ref[0]
    t = cT_ref[0]
    d2 = ((a[:, 0:1] - t[0:1, :]) ** 2
          + (a[:, 1:2] - t[1:2, :]) ** 2
          + (a[:, 2:3] - t[2:3, :]) ** 2)
    iot = lax.broadcasted_iota(jnp.int32, (L, L), 1)
    cols = []
    for _ in range(KNN):
        m = jnp.min(d2, axis=1, keepdims=True)
        am = jnp.min(jnp.where(d2 == m, iot, L), axis=1, keepdims=True)
        cols.append(am)
        d2 = jnp.where(iot == am, jnp.float32(1e30), d2)
    idx_out[0] = jnp.concatenate(cols, axis=1)


def _sc_gather(table, gidx):
    info = plsc.get_sparse_core_info()
    nc, ns = info.num_cores, info.num_subcores
    nw = nc * ns
    rows_w = (KNN * B) // nw
    chunk = 64
    steps = rows_w // chunk

    mesh = plsc.VectorSubcoreMesh(core_axis_name="c", subcore_axis_name="s")

    @functools.partial(
        pl.kernel, mesh=mesh,
        out_type=jax.ShapeDtypeStruct((KNN * B, TW), jnp.float32),
        scratch_types=[
            pltpu.VMEM((chunk,), jnp.int32),
            pltpu.VMEM((chunk, TW), jnp.float32),
            pltpu.SemaphoreType.DMA,
        ],
    )
    def k(table_hbm, gidx_hbm, out_hbm, idx_v, rows_v, sem):
        wid = lax.axis_index("s") * nc + lax.axis_index("c")
        base = wid * rows_w

        def step(i, carry):
            off = base + i * chunk
            pltpu.sync_copy(gidx_hbm.at[pl.ds(off, chunk)], idx_v)
            pltpu.async_copy(table_hbm.at[idx_v], rows_v, sem).wait()
            pltpu.sync_copy(rows_v, out_hbm.at[pl.ds(off, chunk)])
            return carry

        lax.fori_loop(0, steps, step, 0)

    return k(table, gidx)


def _attn(g_ref, feat_ref, pw_ref, seg_ref, segT_ref,
          wposk_ref, wposv_ref, bkk_ref, bvv_ref,
          wq_ref, bq_ref, wo_ref, bo_ref, g1_ref, b1_ref,
          wf1_ref, bf1_ref, wf2_ref, bf2_ref, g2_ref, b2_ref,
          out_ref):
    T = TILE_D
    g = g_ref[:]
    u = lax.bitcast_convert_type(g[:, :, 0:D].reshape(KNN * T, D), jnp.uint32)
    fk = lax.bitcast_convert_type(u & jnp.uint32(0xFFFF0000), jnp.float32)
    fv = lax.bitcast_convert_type(u << 16, jnp.float32)
    pwg = g[:, :, D:D + FH]
    proj = pwg - pw_ref[:][None, :, :]
    tp = (2.0 * jnp.pi) * proj
    ffeat = (jnp.concatenate([jnp.cos(tp), jnp.sin(tp)], axis=-1)
             / jnp.sqrt(jnp.float32(F))).reshape(KNN * T, F)
    kk = (fk + _mm(ffeat, wposk_ref[:]) + bkk_ref[:]).reshape(KNN, T, D)
    vv = (fv + _mm(ffeat, wposv_ref[:]) + bvv_ref[:]).reshape(KNN, T, D)

    tgt = feat_ref[:]
    inv = 1.0 / jnp.sqrt(jnp.float32(DH))
    for _ in range(REPEATS):
        q = _mm(tgt, wq_ref[:]) + bq_ref[:]
        sc = ((kk * q[None, :, :]).reshape(KNN * T, D)
              @ seg_ref[:]).reshape(KNN, T, H) * inv
        m = jnp.max(sc, axis=0, keepdims=True)
        e = jnp.exp(sc - m)
        a = e / jnp.sum(e, axis=0, keepdims=True)
        ab = (a.reshape(KNN * T, H) @ segT_ref[:]).reshape(KNN, T, D)
        att = jnp.sum(ab * vv, axis=0)
        att = _mm(att, wo_ref[:]) + bo_ref[:]
        tgt = _ln(tgt + att, g1_ref[:], b1_ref[:])
        ffn = _mm(_mish(_mm(tgt, wf1_ref[:]) + bf1_ref[:]),
                  wf2_ref[:]) + bf2_ref[:]
        tgt = _ln(tgt + ffn, g2_ref[:], b2_ref[:])
    out_ref[:] = tgt


def kernel(coords, features, Wmlp1, bmlp1, Wmlp2, bmlp2, gmlp, bmlpln,
           Wse1, bse1, Wse2, bse2, gse, bseln, Wr, Wpos, bpos,
           Wq, bq, Wk, bk, Wv, bv, Wo, bo, g1, b1n,
           Wff1, bff1, Wff2, bff2, g2, b2n):
    f32 = jnp.float32
    r2 = lambda v: v.reshape(1, -1)

    feats_flat = features.reshape(B, D_IN)
    c_flat = coords.reshape(B, 3)
    c8 = jnp.pad(c_flat, ((0, 0), (0, 5)))
    wr8 = jnp.pad(Wr, ((0, 5), (0, 0)))
    cn8 = jnp.pad(jnp.transpose(coords, (1, 0, 2)), ((0, 0), (0, 0), (0, 5)))
    cnT = jnp.transpose(cn8, (0, 2, 1))
    nvec = jnp.arange(B, dtype=jnp.int32) % N
    nmask = (nvec[:, None] == jnp.arange(N)[None, :]).astype(f32)
    mmean = nmask.T / f32(L)
    seg = (jnp.arange(D)[:, None] // DH
           == jnp.arange(H)[None, :]).astype(f32)
    segT = seg.T
    wposk = Wpos @ Wk
    wposv = Wpos @ Wv
    bkk = bpos @ Wk + bk
    bvv = bpos @ Wv + bv

    feat, table, pw = pl.pallas_call(
        _featk,
        out_shape=(
            jax.ShapeDtypeStruct((B, D), f32),
            jax.ShapeDtypeStruct((B, TW), f32),
            jax.ShapeDtypeStruct((B, FH), f32),
        ),
    )(feats_flat, c8, mmean, nmask,
      Wmlp1, r2(bmlp1), Wmlp2, r2(bmlp2), r2(gmlp), r2(bmlpln),
      Wse1, r2(bse1), Wse2, r2(bse2), r2(gse), r2(bseln),
      Wk, Wv, wr8)

    idx16 = pl.pallas_call(
        _topk,
        grid=(N,),
        in_specs=[
            pl.BlockSpec((1, L, 8), lambda n: (n, 0, 0)),
            pl.BlockSpec((1, 8, L), lambda n: (n, 0, 0)),
        ],
        out_specs=pl.BlockSpec((1, L, KNN), lambda n: (n, 0, 0)),
        out_shape=jax.ShapeDtypeStruct((N, L, KNN), jnp.int32),
    )(cn8, cnT)

    gidx = (jnp.transpose(idx16, (2, 1, 0)) * N
            + jnp.arange(N, dtype=jnp.int32)[None, None, :]).reshape(KNN * B)

    rows = _sc_gather(table, gidx).reshape(KNN, B, TW)

    nsteps = B // TILE_D
    out = pl.pallas_call(
        _attn,
        grid=(nsteps,),
        in_specs=[
            pl.BlockSpec((KNN, TILE_D, TW), lambda i: (0, i, 0)),
            pl.BlockSpec((TILE_D, D), lambda i: (i, 0)),
            pl.BlockSpec((TILE_D, FH), lambda i: (i, 0)),
            pl.BlockSpec((D, H), lambda i: (0, 0)),
            pl.BlockSpec((H, D), lambda i: (0, 0)),
            pl.BlockSpec((F, D), lambda i: (0, 0)),
            pl.BlockSpec((F, D), lambda i: (0, 0)),
            pl.BlockSpec((1, D), lambda i: (0, 0)),
            pl.BlockSpec((1, D), lambda i: (0, 0)),
            pl.BlockSpec((D, D), lambda i: (0, 0)),
            pl.BlockSpec((1, D), lambda i: (0, 0)),
            pl.BlockSpec((D, D), lambda i: (0, 0)),
            pl.BlockSpec((1, D), lambda i: (0, 0)),
            pl.BlockSpec((1, D), lambda i: (0, 0)),
            pl.BlockSpec((1, D), lambda i: (0, 0)),
            pl.BlockSpec((D, 4 * D), lambda i: (0, 0)),
            pl.BlockSpec((1, 4 * D), lambda i: (0, 0)),
            pl.BlockSpec((4 * D, D), lambda i: (0, 0)),
            pl.BlockSpec((1, D), lambda i: (0, 0)),
            pl.BlockSpec((1, D), lambda i: (0, 0)),
            pl.BlockSpec((1, D), lambda i: (0, 0)),
        ],
        out_specs=pl.BlockSpec((TILE_D, D), lambda i: (i, 0)),
        out_shape=jax.ShapeDtypeStruct((B, D), f32),
    )(rows, feat, pw, seg, segT, wposk, wposv, r2(bkk), r2(bvv),
      Wq, r2(bq), Wo, r2(bo), r2(g1), r2(b1n),
      Wff1, r2(bff1), Wff2, r2(bff2), r2(g2), r2(b2n))

    return out.reshape(L, N, D)


# speedup vs baseline: 5.7260x; 1.0261x over previous
import functools

import jax
import jax.numpy as jnp
from jax import lax
from jax.experimental import pallas as pl
from jax.experimental.pallas import tpu as pltpu
from jax.experimental.pallas import tpu_sc as plsc

L, N, D_IN, D, F, KNN, H, REPEATS = 1024, 4, 64, 256, 128, 16, 4, 2
B = L * N
DH = D // H
FH = F // 2
TW = 384
TILE_D = 128


def _mm(a, b):
    return lax.dot(a.astype(jnp.bfloat16), b.astype(jnp.bfloat16),
                   preferred_element_type=jnp.float32)


def _mish(x):
    sp = jnp.maximum(x, 0.0) + jnp.log1p(jnp.exp(-jnp.abs(x)))
    return x * jnp.tanh(sp)


def _ln(x, g, b, eps=1e-5):
    m = x.mean(-1, keepdims=True)
    v = ((x - m) ** 2).mean(-1, keepdims=True)
    return (x - m) / jnp.sqrt(v + eps) * g + b


def _featk(feats_ref, c8_ref, mmean_ref, nmask_ref,
           w1_ref, b1_ref, w2_ref, b2_ref, gm_ref, bm_ref,
           ws1_ref, bs1_ref, ws2_ref, bs2_ref, gs_ref, bs_ref,
           wk_ref, wv_ref, wr8_ref,
           feat_out, table_out, pw_out):
    x = feats_ref[:]
    h = _mish(x @ w1_ref[:] + b1_ref[:]) @ w2_ref[:] + b2_ref[:]
    feat = _ln(h, gm_ref[:], bm_ref[:])
    s4 = mmean_ref[:] @ feat
    s4 = jax.nn.sigmoid(
        jnp.maximum(s4 @ ws1_ref[:] + bs1_ref[:], 0.0) @ ws2_ref[:] + bs2_ref[:])
    srow = nmask_ref[:] @ s4
    feat = _ln(feat + feat * srow, gs_ref[:], bs_ref[:])
    fk = feat @ wk_ref[:]
    fv = feat @ wv_ref[:]
    pw = c8_ref[:] @ wr8_ref[:]
    feat_out[:] = feat
    pw_out[:] = pw
    uk = lax.bitcast_convert_type(fk, jnp.uint32)
    uv = lax.bitcast_convert_type(fv, jnp.uint32)
    packed = lax.bitcast_convert_type(
        (uk & jnp.uint32(0xFFFF0000)) | (uv >> 16), jnp.float32)
    table_out[:] = jnp.concatenate(
        [packed, pw, jnp.zeros((B, TW - D - FH), jnp.float32)], axis=1)


def _topk(c8_ref, cT_ref, idx_out):
    a = c8_ref[0]
    t = cT_ref[0]
    d2 = ((a[:, 0:1] - t[0:1, :]) ** 2
          + (a[:, 1:2] - t[1:2, :]) ** 2
          + (a[:, 2:3] - t[2:3, :]) ** 2)
    iot = lax.broadcasted_iota(jnp.int32, (L, L), 1)
    cols = []
    for _ in range(KNN):
        am = jnp.argmin(d2, axis=1).astype(jnp.int32)[:, None]
        cols.append(am)
        d2 = jnp.where(iot == am, jnp.float32(1e30), d2)
    idx_out[0] = jnp.concatenate(cols, axis=1)


def _sc_gather(table, gidx):
    info = plsc.get_sparse_core_info()
    nc, ns = info.num_cores, info.num_subcores
    nw = nc * ns
    rows_w = (KNN * B) // nw
    chunk = 64
    steps = rows_w // chunk

    mesh = plsc.VectorSubcoreMesh(core_axis_name="c", subcore_axis_name="s")

    @functools.partial(
        pl.kernel, mesh=mesh,
        out_type=jax.ShapeDtypeStruct((KNN * B, TW), jnp.float32),
        scratch_types=[
            pltpu.VMEM((chunk,), jnp.int32),
            pltpu.VMEM((chunk, TW), jnp.float32),
            pltpu.SemaphoreType.DMA,
        ],
    )
    def k(table_hbm, gidx_hbm, out_hbm, idx_v, rows_v, sem):
        wid = lax.axis_index("s") * nc + lax.axis_index("c")
        base = wid * rows_w

        def step(i, carry):
            off = base + i * chunk
            pltpu.sync_copy(gidx_hbm.at[pl.ds(off, chunk)], idx_v)
            pltpu.async_copy(table_hbm.at[idx_v], rows_v, sem).wait()
            pltpu.sync_copy(rows_v, out_hbm.at[pl.ds(off, chunk)])
            return carry

        lax.fori_loop(0, steps, step, 0)

    return k(table, gidx)


def _attn(g_ref, feat_ref, pw_ref, seg_ref, segT_ref,
          wposk_ref, wposv_ref, bkk_ref, bvv_ref,
          wq_ref, bq_ref, wo_ref, bo_ref, g1_ref, b1_ref,
          wf1_ref, bf1_ref, wf2_ref, bf2_ref, g2_ref, b2_ref,
          out_ref):
    T = TILE_D
    g = g_ref[:]
    u = lax.bitcast_convert_type(g[:, :, 0:D].reshape(KNN * T, D), jnp.uint32)
    fk = lax.bitcast_convert_type(u & jnp.uint32(0xFFFF0000), jnp.float32)
    fv = lax.bitcast_convert_type(u << 16, jnp.float32)
    pwg = g[:, :, D:D + FH]
    proj = pwg - pw_ref[:][None, :, :]
    tp = (2.0 * jnp.pi) * proj
    ffeat = (jnp.concatenate([jnp.cos(tp), jnp.sin(tp)], axis=-1)
             / jnp.sqrt(jnp.float32(F))).reshape(KNN * T, F)
    kk = (fk + _mm(ffeat, wposk_ref[:]) + bkk_ref[:]).reshape(KNN, T, D)
    vv = (fv + _mm(ffeat, wposv_ref[:]) + bvv_ref[:]).reshape(KNN, T, D)

    tgt = feat_ref[:]
    inv = 1.0 / jnp.sqrt(jnp.float32(DH))
    for _ in range(REPEATS):
        q = _mm(tgt, wq_ref[:]) + bq_ref[:]
        sc = ((kk * q[None, :, :]).reshape(KNN * T, D)
              @ seg_ref[:]).reshape(KNN, T, H) * inv
        m = jnp.max(sc, axis=0, keepdims=True)
        e = jnp.exp(sc - m)
        a = e / jnp.sum(e, axis=0, keepdims=True)
        ab = (a.reshape(KNN * T, H) @ segT_ref[:]).reshape(KNN, T, D)
        att = jnp.sum(ab * vv, axis=0)
        att = _mm(att, wo_ref[:]) + bo_ref[:]
        tgt = _ln(tgt + att, g1_ref[:], b1_ref[:])
        ffn = _mm(_mish(_mm(tgt, wf1_ref[:]) + bf1_ref[:]),
                  wf2_ref[:]) + bf2_ref[:]
        tgt = _ln(tgt + ffn, g2_ref[:], b2_ref[:])
    out_ref[:] = tgt


def kernel(coords, features, Wmlp1, bmlp1, Wmlp2, bmlp2, gmlp, bmlpln,
           Wse1, bse1, Wse2, bse2, gse, bseln, Wr, Wpos, bpos,
           Wq, bq, Wk, bk, Wv, bv, Wo, bo, g1, b1n,
           Wff1, bff1, Wff2, bff2, g2, b2n):
    f32 = jnp.float32
    r2 = lambda v: v.reshape(1, -1)

    feats_flat = features.reshape(B, D_IN)
    c_flat = coords.reshape(B, 3)
    c8 = jnp.pad(c_flat, ((0, 0), (0, 5)))
    wr8 = jnp.pad(Wr, ((0, 5), (0, 0)))
    cn8 = jnp.pad(jnp.transpose(coords, (1, 0, 2)), ((0, 0), (0, 0), (0, 5)))
    cnT = jnp.transpose(cn8, (0, 2, 1))
    nvec = jnp.arange(B, dtype=jnp.int32) % N
    nmask = (nvec[:, None] == jnp.arange(N)[None, :]).astype(f32)
    mmean = nmask.T / f32(L)
    seg = (jnp.arange(D)[:, None] // DH
           == jnp.arange(H)[None, :]).astype(f32)
    segT = seg.T
    wposk = Wpos @ Wk
    wposv = Wpos @ Wv
    bkk = bpos @ Wk + bk
    bvv = bpos @ Wv + bv

    feat, table, pw = pl.pallas_call(
        _featk,
        out_shape=(
            jax.ShapeDtypeStruct((B, D), f32),
            jax.ShapeDtypeStruct((B, TW), f32),
            jax.ShapeDtypeStruct((B, FH), f32),
        ),
    )(feats_flat, c8, mmean, nmask,
      Wmlp1, r2(bmlp1), Wmlp2, r2(bmlp2), r2(gmlp), r2(bmlpln),
      Wse1, r2(bse1), Wse2, r2(bse2), r2(gse), r2(bseln),
      Wk, Wv, wr8)

    idx16 = pl.pallas_call(
        _topk,
        grid=(N,),
        in_specs=[
            pl.BlockSpec((1, L, 8), lambda n: (n, 0, 0)),
            pl.BlockSpec((1, 8, L), lambda n: (n, 0, 0)),
        ],
        out_specs=pl.BlockSpec((1, L, KNN), lambda n: (n, 0, 0)),
        out_shape=jax.ShapeDtypeStruct((N, L, KNN), jnp.int32),
    )(cn8, cnT)

    gidx = (jnp.transpose(idx16, (2, 1, 0)) * N
            + jnp.arange(N, dtype=jnp.int32)[None, None, :]).reshape(KNN * B)

    rows = _sc_gather(table, gidx).reshape(KNN, B, TW)

    nsteps = B // TILE_D
    out = pl.pallas_call(
        _attn,
        grid=(nsteps,),
        in_specs=[
            pl.BlockSpec((KNN, TILE_D, TW), lambda i: (0, i, 0)),
            pl.BlockSpec((TILE_D, D), lambda i: (i, 0)),
            pl.BlockSpec((TILE_D, FH), lambda i: (i, 0)),
            pl.BlockSpec((D, H), lambda i: (0, 0)),
            pl.BlockSpec((H, D), lambda i: (0, 0)),
            pl.BlockSpec((F, D), lambda i: (0, 0)),
            pl.BlockSpec((F, D), lambda i: (0, 0)),
            pl.BlockSpec((1, D), lambda i: (0, 0)),
            pl.BlockSpec((1, D), lambda i: (0, 0)),
            pl.BlockSpec((D, D), lambda i: (0, 0)),
            pl.BlockSpec((1, D), lambda i: (0, 0)),
            pl.BlockSpec((D, D), lambda i: (0, 0)),
            pl.BlockSpec((1, D), lambda i: (0, 0)),
            pl.BlockSpec((1, D), lambda i: (0, 0)),
            pl.BlockSpec((1, D), lambda i: (0, 0)),
            pl.BlockSpec((D, 4 * D), lambda i: (0, 0)),
            pl.BlockSpec((1, 4 * D), lambda i: (0, 0)),
            pl.BlockSpec((4 * D, D), lambda i: (0, 0)),
            pl.BlockSpec((1, D), lambda i: (0, 0)),
            pl.BlockSpec((1, D), lambda i: (0, 0)),
            pl.BlockSpec((1, D), lambda i: (0, 0)),
        ],
        out_specs=pl.BlockSpec((TILE_D, D), lambda i: (i, 0)),
        out_shape=jax.ShapeDtypeStruct((B, D), f32),
    )(rows, feat, pw, seg, segT, wposk, wposv, r2(bkk), r2(bvv),
      Wq, r2(bq), Wo, r2(bo), r2(g1), r2(b1n),
      Wff1, r2(bff1), Wff2, r2(bff2), r2(g2), r2(b2n))

    return out.reshape(L, N, D)


# speedup vs baseline: 6.7924x; 1.1862x over previous
import functools

import jax
import jax.numpy as jnp
from jax import lax
from jax.experimental import pallas as pl
from jax.experimental.pallas import tpu as pltpu
from jax.experimental.pallas import tpu_sc as plsc

L, N, D_IN, D, F, KNN, H, REPEATS = 1024, 4, 64, 256, 128, 16, 4, 2
B = L * N
DH = D // H
FH = F // 2
TW = 384
TILE_D = 128
SPLIT = 4


def _mm(a, b):
    return lax.dot(a.astype(jnp.bfloat16), b.astype(jnp.bfloat16),
                   preferred_element_type=jnp.float32)


def _mish(x):
    sp = jnp.maximum(x, 0.0) + jnp.log1p(jnp.exp(-jnp.abs(x)))
    return x * jnp.tanh(sp)


def _ln(x, g, b, eps=1e-5):
    m = x.mean(-1, keepdims=True)
    v = ((x - m) ** 2).mean(-1, keepdims=True)
    return (x - m) / jnp.sqrt(v + eps) * g + b


def _featk(feats_ref, c8_ref, mmean_ref, nmask_ref,
           w1_ref, b1_ref, w2_ref, b2_ref, gm_ref, bm_ref,
           ws1_ref, bs1_ref, ws2_ref, bs2_ref, gs_ref, bs_ref,
           wk_ref, wv_ref, wr8_ref,
           feat_out, table_out, pw_out):
    x = feats_ref[:]
    h = _mish(x @ w1_ref[:] + b1_ref[:]) @ w2_ref[:] + b2_ref[:]
    feat = _ln(h, gm_ref[:], bm_ref[:])
    s4 = mmean_ref[:] @ feat
    s4 = jax.nn.sigmoid(
        jnp.maximum(s4 @ ws1_ref[:] + bs1_ref[:], 0.0) @ ws2_ref[:] + bs2_ref[:])
    srow = nmask_ref[:] @ s4
    feat = _ln(feat + feat * srow, gs_ref[:], bs_ref[:])
    fk = feat @ wk_ref[:]
    fv = feat @ wv_ref[:]
    pw = c8_ref[:] @ wr8_ref[:]
    feat_out[:] = feat
    pw_out[:] = pw
    uk = lax.bitcast_convert_type(fk, jnp.uint32)
    uv = lax.bitcast_convert_type(fv, jnp.uint32)
    packed = lax.bitcast_convert_type(
        (uk & jnp.uint32(0xFFFF0000)) | (uv >> 16), jnp.float32)
    pieces = [packed, pw]
    if TW > D + FH:
        pieces.append(jnp.zeros((B, TW - D - FH), jnp.float32))
    table_out[:] = jnp.concatenate(pieces, axis=1)


def _topk(c8_ref, cT_ref, idx_out):
    a = c8_ref[0]
    t = cT_ref[0]
    d2 = ((a[:, 0:1] - t[0:1, :]) ** 2
          + (a[:, 1:2] - t[1:2, :]) ** 2
          + (a[:, 2:3] - t[2:3, :]) ** 2)
    iot = lax.broadcasted_iota(jnp.int32, (L, L), 1)
    cols = []
    for _ in range(KNN):
        am = jnp.argmin(d2, axis=1).astype(jnp.int32)[:, None]
        cols.append(am)
        d2 = jnp.where(iot == am, jnp.float32(1e30), d2)
    idx_out[0] = jnp.concatenate(cols, axis=1)


def _sc_gather(table, gidx, nrows):
    info = plsc.get_sparse_core_info()
    nc, ns = info.num_cores, info.num_subcores
    nw = nc * ns
    rows_w = nrows // nw
    chunk = 64
    steps = rows_w // chunk

    mesh = plsc.VectorSubcoreMesh(core_axis_name="c", subcore_axis_name="s")

    @functools.partial(
        pl.kernel, mesh=mesh,
        out_type=jax.ShapeDtypeStruct((nrows, TW), jnp.float32),
        scratch_types=[
            pltpu.VMEM((chunk,), jnp.int32),
            pltpu.VMEM((chunk, TW), jnp.float32),
            pltpu.SemaphoreType.DMA,
        ],
    )
    def k(table_hbm, gidx_hbm, out_hbm, idx_v, rows_v, sem):
        wid = lax.axis_index("s") * nc + lax.axis_index("c")
        base = wid * rows_w

        def step(i, carry):
            off = base + i * chunk
            pltpu.sync_copy(gidx_hbm.at[pl.ds(off, chunk)], idx_v)
            pltpu.async_copy(table_hbm.at[idx_v], rows_v, sem).wait()
            pltpu.sync_copy(rows_v, out_hbm.at[pl.ds(off, chunk)])
            return carry

        lax.fori_loop(0, steps, step, 0)

    return k(table, gidx)


def _attn(g_ref, feat_ref, pw_ref, seg_ref, segT_ref,
          wposk_ref, wposv_ref, bkk_ref, bvv_ref,
          wq_ref, bq_ref, wo_ref, bo_ref, g1_ref, b1_ref,
          wf1_ref, bf1_ref, wf2_ref, bf2_ref, g2_ref, b2_ref,
          out_ref):
    T = TILE_D
    g = g_ref[:]
    u = lax.bitcast_convert_type(g[:, :, 0:D].reshape(T * KNN, D), jnp.uint32)
    fk = lax.bitcast_convert_type(u & jnp.uint32(0xFFFF0000), jnp.float32)
    fv = lax.bitcast_convert_type(u << 16, jnp.float32)
    pwg = g[:, :, D:D + FH]
    proj = pwg - pw_ref[:][:, None, :]
    tp = (2.0 * jnp.pi) * proj
    ffeat = (jnp.concatenate([jnp.cos(tp), jnp.sin(tp)], axis=-1)
             / jnp.sqrt(jnp.float32(F))).reshape(T * KNN, F)
    kk = (fk + _mm(ffeat, wposk_ref[:]) + bkk_ref[:]).reshape(T, KNN, D)
    vv = (fv + _mm(ffeat, wposv_ref[:]) + bvv_ref[:]).reshape(T, KNN, D)

    tgt = feat_ref[:]
    inv = 1.0 / jnp.sqrt(jnp.float32(DH))
    for _ in range(REPEATS):
        q = _mm(tgt, wq_ref[:]) + bq_ref[:]
        sc = ((kk * q[:, None, :]).reshape(T * KNN, D)
              @ seg_ref[:]).reshape(T, KNN, H) * inv
        m = jnp.max(sc, axis=1, keepdims=True)
        e = jnp.exp(sc - m)
        a = e / jnp.sum(e, axis=1, keepdims=True)
        ab = (a.reshape(T * KNN, H) @ segT_ref[:]).reshape(T, KNN, D)
        att = jnp.sum(ab * vv, axis=1)
        att = _mm(att, wo_ref[:]) + bo_ref[:]
        tgt = _ln(tgt + att, g1_ref[:], b1_ref[:])
        ffn = _mm(_mish(_mm(tgt, wf1_ref[:]) + bf1_ref[:]),
                  wf2_ref[:]) + bf2_ref[:]
        tgt = _ln(tgt + ffn, g2_ref[:], b2_ref[:])
    out_ref[:] = tgt


def kernel(coords, features, Wmlp1, bmlp1, Wmlp2, bmlp2, gmlp, bmlpln,
           Wse1, bse1, Wse2, bse2, gse, bseln, Wr, Wpos, bpos,
           Wq, bq, Wk, bk, Wv, bv, Wo, bo, g1, b1n,
           Wff1, bff1, Wff2, bff2, g2, b2n):
    f32 = jnp.float32
    r2 = lambda v: v.reshape(1, -1)

    feats_flat = features.reshape(B, D_IN)
    c_flat = coords.reshape(B, 3)
    c8 = jnp.pad(c_flat, ((0, 0), (0, 5)))
    wr8 = jnp.pad(Wr, ((0, 5), (0, 0)))
    cn8 = jnp.pad(jnp.transpose(coords, (1, 0, 2)), ((0, 0), (0, 0), (0, 5)))
    cnT = jnp.transpose(cn8, (0, 2, 1))
    nvec = jnp.arange(B, dtype=jnp.int32) % N
    nmask = (nvec[:, None] == jnp.arange(N)[None, :]).astype(f32)
    mmean = nmask.T / f32(L)
    seg = (jnp.arange(D)[:, None] // DH
           == jnp.arange(H)[None, :]).astype(f32)
    segT = seg.T
    wposk = Wpos @ Wk
    wposv = Wpos @ Wv
    bkk = bpos @ Wk + bk
    bvv = bpos @ Wv + bv

    feat, table, pw = pl.pallas_call(
        _featk,
        out_shape=(
            jax.ShapeDtypeStruct((B, D), f32),
            jax.ShapeDtypeStruct((B, TW), f32),
            jax.ShapeDtypeStruct((B, FH), f32),
        ),
    )(feats_flat, c8, mmean, nmask,
      Wmlp1, r2(bmlp1), Wmlp2, r2(bmlp2), r2(gmlp), r2(bmlpln),
      Wse1, r2(bse1), Wse2, r2(bse2), r2(gse), r2(bseln),
      Wk, Wv, wr8)

    idx16 = pl.pallas_call(
        _topk,
        grid=(N,),
        in_specs=[
            pl.BlockSpec((1, L, 8), lambda n: (n, 0, 0)),
            pl.BlockSpec((1, 8, L), lambda n: (n, 0, 0)),
        ],
        out_specs=pl.BlockSpec((1, L, KNN), lambda n: (n, 0, 0)),
        out_shape=jax.ShapeDtypeStruct((N, L, KNN), jnp.int32),
    )(cn8, cnT)

    gidx = (jnp.transpose(idx16, (1, 0, 2)) * N
            + jnp.arange(N, dtype=jnp.int32)[None, :, None]).reshape(KNN * B)

    bs = B // SPLIT
    wts = (seg, segT, wposk, wposv, r2(bkk), r2(bvv),
           Wq, r2(bq), Wo, r2(bo), r2(g1), r2(b1n),
           Wff1, r2(bff1), Wff2, r2(bff2), r2(g2), r2(b2n))
    wspecs = [
        pl.BlockSpec((D, H), lambda i: (0, 0)),
        pl.BlockSpec((H, D), lambda i: (0, 0)),
        pl.BlockSpec((F, D), lambda i: (0, 0)),
        pl.BlockSpec((F, D), lambda i: (0, 0)),
        pl.BlockSpec((1, D), lambda i: (0, 0)),
        pl.BlockSpec((1, D), lambda i: (0, 0)),
        pl.BlockSpec((D, D), lambda i: (0, 0)),
        pl.BlockSpec((1, D), lambda i: (0, 0)),
        pl.BlockSpec((D, D), lambda i: (0, 0)),
        pl.BlockSpec((1, D), lambda i: (0, 0)),
        pl.BlockSpec((1, D), lambda i: (0, 0)),
        pl.BlockSpec((1, D), lambda i: (0, 0)),
        pl.BlockSpec((D, 4 * D), lambda i: (0, 0)),
        pl.BlockSpec((1, 4 * D), lambda i: (0, 0)),
        pl.BlockSpec((4 * D, D), lambda i: (0, 0)),
        pl.BlockSpec((1, D), lambda i: (0, 0)),
        pl.BlockSpec((1, D), lambda i: (0, 0)),
        pl.BlockSpec((1, D), lambda i: (0, 0)),
    ]
    gidx3 = gidx.reshape(SPLIT, bs * KNN)
    feat3 = feat.reshape(SPLIT, bs, D)
    pw3 = pw.reshape(SPLIT, bs, FH)
    parts = []
    for s in range(SPLIT):
        rows_s = _sc_gather(table, gidx3[s], bs * KNN).reshape(bs, KNN, TW)
        part = pl.pallas_call(
            _attn,
            grid=(bs // TILE_D,),
            in_specs=[
                pl.BlockSpec((TILE_D, KNN, TW), lambda i: (i, 0, 0)),
                pl.BlockSpec((TILE_D, D), lambda i: (i, 0)),
                pl.BlockSpec((TILE_D, FH), lambda i: (i, 0)),
            ] + wspecs,
            out_specs=pl.BlockSpec((TILE_D, D), lambda i: (i, 0)),
            out_shape=jax.ShapeDtypeStruct((bs, D), f32),
        )(rows_s, feat3[s], pw3[s], *wts)
        parts.append(part)

    return jnp.concatenate(parts, axis=0).reshape(L, N, D)
